# Initial kernel scaffold; baseline (speedup 1.0000x reference)
import jax
import jax.numpy as jnp
from jax.experimental import pallas as pl


def kernel(x, edge_index, training, W_out0, b_out0, W_root0, W_out1, b_out1, W_root1, W_out2, b_out2, W_root2):
    raise NotImplementedError("write your pallas kernel here")


# speedup vs baseline: 7.1811x; 7.1811x over previous
import functools

import jax
import jax.numpy as jnp
from jax import lax
from jax.experimental import pallas as pl
from jax.experimental.pallas import tpu as pltpu
from jax.experimental.pallas import tpu_sc as plsc

N = 10000
E = 320000
D = 128

NC = 2
NS = 16
NW = NC * NS
B = 128

E_PAD = ((E + NW * B - 1) // (NW * B)) * (NW * B)
EPT = E_PAD // NW
NB = EPT // B
N_PAD = 10112
RPT = N_PAD // NS
DEG_W = 128

_MESH = plsc.VectorSubcoreMesh(
    core_axis_name="c", subcore_axis_name="s", num_cores=NC, num_subcores=NS
)


@functools.partial(
    pl.kernel,
    out_type=jax.ShapeDtypeStruct((NC, N_PAD, D), jnp.float32),
    mesh=_MESH,
    scratch_types=[
        pltpu.VMEM((B,), jnp.int32),
        pltpu.VMEM((B,), jnp.int32),
        pltpu.VMEM((B, D), jnp.float32),
        pltpu.VMEM_SHARED((N_PAD, D), jnp.float32),
        pltpu.SemaphoreType.DMA,
    ],
)
def _sc_edge_pass(h_hbm, src_hbm, dst_hbm, zeros_hbm, out_hbm,
                  src_v, dst_v, rows_v, acc_sh, sem):
    c = lax.axis_index("c")
    s = lax.axis_index("s")
    wid = s * NC + c
    row0 = pl.multiple_of(s * RPT, 8)
    pltpu.sync_copy(zeros_hbm.at[pl.ds(row0, RPT)], acc_sh.at[pl.ds(row0, RPT)])
    plsc.subcore_barrier()
    base = wid * EPT

    def step(i, carry):
        off = pl.multiple_of(base + i * B, B)
        pltpu.sync_copy(src_hbm.at[pl.ds(off, B)], src_v)
        pltpu.sync_copy(dst_hbm.at[pl.ds(off, B)], dst_v)
        pltpu.async_copy(h_hbm.at[src_v], rows_v, sem).wait()
        pltpu.sync_copy(rows_v, acc_sh.at[dst_v], add=True)
        return carry

    lax.fori_loop(0, NB, step, 0)
    plsc.subcore_barrier()
    pltpu.sync_copy(acc_sh.at[pl.ds(row0, RPT)], out_hbm.at[c, pl.ds(row0, RPT)])


@functools.partial(
    pl.kernel,
    out_type=jax.ShapeDtypeStruct((NC, N_PAD, DEG_W), jnp.float32),
    mesh=_MESH,
    scratch_types=[
        pltpu.VMEM((B,), jnp.int32),
        pltpu.VMEM((B, DEG_W), jnp.float32),
        pltpu.VMEM_SHARED((N_PAD, DEG_W), jnp.float32),
    ],
)
def _sc_deg_pass(dst_hbm, zeros_hbm, ones_hbm, out_hbm,
                 dst_v, ones_v, acc_sh):
    c = lax.axis_index("c")
    s = lax.axis_index("s")
    wid = s * NC + c
    row0 = pl.multiple_of(s * RPT, 8)
    pltpu.sync_copy(zeros_hbm.at[pl.ds(row0, RPT)], acc_sh.at[pl.ds(row0, RPT)])
    pltpu.sync_copy(ones_hbm, ones_v)
    plsc.subcore_barrier()
    base = wid * EPT

    def step(i, carry):
        off = pl.multiple_of(base + i * B, B)
        pltpu.sync_copy(dst_hbm.at[pl.ds(off, B)], dst_v)
        pltpu.sync_copy(ones_v, acc_sh.at[dst_v], add=True)
        return carry

    lax.fori_loop(0, NB, step, 0)
    plsc.subcore_barrier()
    pltpu.sync_copy(acc_sh.at[pl.ds(row0, RPT)], out_hbm.at[c, pl.ds(row0, RPT)])


_TC_R = 1000


def _tc_layer_body(relu, h_ref, p_ref, dp_ref, wo_ref, b_ref, wr_ref, o_ref):
    hb = h_ref[...]
    agg = hb + p_ref[0] + p_ref[1]
    deg = 1.0 + dp_ref[0] + dp_ref[1]
    agg = agg / jnp.maximum(deg, 1.0)
    out = (
        jnp.dot(agg, wo_ref[...], preferred_element_type=jnp.float32)
        + jnp.dot(hb, wr_ref[...], preferred_element_type=jnp.float32)
        + b_ref[...]
    )
    o_ref[...] = jnp.maximum(out, 0.0) if relu else out


def _tc_layer(h, p, dp, w_out, b_out, w_root, relu):
    grid = (N // _TC_R,)
    return pl.pallas_call(
        functools.partial(_tc_layer_body, relu),
        grid=grid,
        in_specs=[
            pl.BlockSpec((_TC_R, D), lambda i: (i, 0)),
            pl.BlockSpec((NC, _TC_R, D), lambda i: (0, i, 0)),
            pl.BlockSpec((NC, _TC_R, DEG_W), lambda i: (0, i, 0)),
            pl.BlockSpec((D, D), lambda i: (0, 0)),
            pl.BlockSpec((1, D), lambda i: (0, 0)),
            pl.BlockSpec((D, D), lambda i: (0, 0)),
        ],
        out_specs=pl.BlockSpec((_TC_R, D), lambda i: (i, 0)),
        out_shape=jax.ShapeDtypeStruct((N, D), jnp.float32),
    )(h, p, dp, w_out, b_out, w_root)


def kernel(x, edge_index, training,
           W_out0, b_out0, W_root0,
           W_out1, b_out1, W_root1,
           W_out2, b_out2, W_root2):
    del training
    src = edge_index[0]
    dst = edge_index[1]
    pad_e = E_PAD - E
    src_p = jnp.concatenate([src, jnp.zeros((pad_e,), jnp.int32)])
    dst_p = jnp.concatenate([dst, jnp.full((pad_e,), N, jnp.int32)])
    zeros_tbl = jnp.zeros((N_PAD, D), jnp.float32)
    zeros_deg = jnp.zeros((N_PAD, DEG_W), jnp.float32)
    ones_blk = jnp.ones((B, DEG_W), jnp.float32)

    dp = _sc_deg_pass(dst_p, zeros_deg, ones_blk)

    b0 = b_out0.reshape(1, D)
    b1 = b_out1.reshape(1, D)
    b2 = b_out2.reshape(1, D)

    h = x
    p = _sc_edge_pass(h, src_p, dst_p, zeros_tbl)
    h = _tc_layer(h, p, dp, W_out0, b0, W_root0, relu=True)
    p = _sc_edge_pass(h, src_p, dst_p, zeros_tbl)
    h = _tc_layer(h, p, dp, W_out1, b1, W_root1, relu=True)
    p = _sc_edge_pass(h, src_p, dst_p, zeros_tbl)
    return _tc_layer(h, p, dp, W_out2, b2, W_root2, relu=False)


# speedup vs baseline: 21.0711x; 2.9343x over previous
import functools

import jax
import jax.numpy as jnp
from jax import lax
from jax.experimental import pallas as pl
from jax.experimental.pallas import tpu as pltpu
from jax.experimental.pallas import tpu_sc as plsc

N = 10000
E = 320000
D = 128

NC = 2
NS = 16
NW = NC * NS
B = 128
NB = 80
NBUF = 2
CH = 8
NCH = NB // CH

EPT = NB * B
E_PAD = NW * EPT
N_PAD = 10112
RPT = N_PAD // NS

_MESH = plsc.VectorSubcoreMesh(
    core_axis_name="c", subcore_axis_name="s", num_cores=NC, num_subcores=NS
)


@functools.partial(
    pl.kernel,
    out_type=jax.ShapeDtypeStruct((NC, N_PAD, D), jnp.float32),
    mesh=_MESH,
    scratch_types=[
        pltpu.VMEM((3, CH, B), jnp.int32),
        pltpu.VMEM((3, CH, B), jnp.int32),
        pltpu.VMEM((NBUF, B, D), jnp.float32),
        pltpu.VMEM_SHARED((N_PAD, D), jnp.float32),
        pltpu.SemaphoreType.DMA,
        pltpu.SemaphoreType.DMA,
        pltpu.SemaphoreType.DMA,
    ],
)
def _sc_edge_pass(h_hbm, src_hbm, dst_hbm, zeros_hbm, out_hbm,
                  src_ch, dst_ch, rows, acc_sh, isem, gsem, ssem):
    c = lax.axis_index("c")
    s = lax.axis_index("s")
    wid = s * NC + c
    row0 = pl.multiple_of(s * RPT, 8)
    pltpu.sync_copy(zeros_hbm.at[pl.ds(row0, RPT)], acc_sh.at[pl.ds(row0, RPT)])

    def i_start(k):
        slot = lax.rem(k, 3)
        pltpu.async_copy(src_hbm.at[wid, pl.ds(k * CH, CH)], src_ch.at[slot], isem)
        pltpu.async_copy(dst_hbm.at[wid, pl.ds(k * CH, CH)], dst_ch.at[slot], isem)

    def i_wait(k):
        slot = lax.rem(k, 3)
        pltpu.make_async_copy(
            src_hbm.at[wid, pl.ds(k * CH, CH)], src_ch.at[slot], isem).wait()
        pltpu.make_async_copy(
            dst_hbm.at[wid, pl.ds(k * CH, CH)], dst_ch.at[slot], isem).wait()

    def _slot_jj(i):
        k = lax.div(i, CH)
        return lax.rem(k, 3), lax.rem(i, CH)

    def g_start(i):
        slot, jj = _slot_jj(i)
        pltpu.async_copy(
            h_hbm.at[src_ch.at[slot, jj]], rows.at[lax.rem(i, NBUF)], gsem)

    def g_wait(i):
        slot, jj = _slot_jj(i)
        pltpu.make_async_copy(
            h_hbm.at[src_ch.at[slot, jj]], rows.at[lax.rem(i, NBUF)], gsem).wait()

    def s_start(i):
        slot, jj = _slot_jj(i)
        pltpu.async_copy(
            rows.at[lax.rem(i, NBUF)], acc_sh.at[dst_ch.at[slot, jj]], ssem,
            add=True)

    def s_wait(i):
        slot, jj = _slot_jj(i)
        pltpu.make_async_copy(
            rows.at[lax.rem(i, NBUF)], acc_sh.at[dst_ch.at[slot, jj]], ssem).wait()

    i_start(0)
    i_wait(0)
    i_start(1)
    for i in range(NBUF):
        g_start(i)
    plsc.subcore_barrier()

    def body(i, carry):
        g_wait(i)
        s_start(i)
        s_wait(i)

        def lookahead():
            nxt = i + NBUF

            def chunk_turn():
                k = lax.div(nxt, CH)
                i_wait(k)
                pl.when(k + 1 < NCH)(lambda: i_start(k + 1))

            pl.when(lax.rem(nxt, CH) == 0)(chunk_turn)
            g_start(nxt)

        pl.when(i + NBUF < NB)(lookahead)
        return carry

    lax.fori_loop(0, NB, body, 0)

    plsc.subcore_barrier()
    pltpu.sync_copy(acc_sh.at[pl.ds(row0, RPT)], out_hbm.at[c, pl.ds(row0, RPT)])


@functools.partial(
    pl.kernel,
    out_type=jax.ShapeDtypeStruct((NC, N_PAD, D), jnp.float32),
    mesh=_MESH,
    scratch_types=[
        pltpu.VMEM((NB, B), jnp.int32),
        pltpu.VMEM((B, D), jnp.float32),
        pltpu.VMEM_SHARED((N_PAD, D), jnp.float32),
        pltpu.SemaphoreType.DMA,
    ],
)
def _sc_deg_pass(dst_hbm, zeros_hbm, ones_hbm, out_hbm,
                 dst_all, ones_v, acc_sh, ssem):
    c = lax.axis_index("c")
    s = lax.axis_index("s")
    wid = s * NC + c
    row0 = pl.multiple_of(s * RPT, 8)
    pltpu.sync_copy(zeros_hbm.at[pl.ds(row0, RPT)], acc_sh.at[pl.ds(row0, RPT)])
    pltpu.sync_copy(dst_hbm.at[wid], dst_all)
    pltpu.sync_copy(ones_hbm, ones_v)
    plsc.subcore_barrier()

    def s_start(i):
        pltpu.async_copy(ones_v, acc_sh.at[dst_all.at[i]], ssem, add=True)

    def s_wait(i):
        pltpu.make_async_copy(ones_v, acc_sh.at[dst_all.at[i]], ssem).wait()

    def body(i, carry):
        s_start(i)
        pl.when(i >= 3)(lambda: s_wait(i - 3))
        return carry

    lax.fori_loop(0, NB, body, 0)
    for i in range(NB - 3, NB):
        s_wait(i)

    plsc.subcore_barrier()
    pltpu.sync_copy(acc_sh.at[pl.ds(row0, RPT)], out_hbm.at[c, pl.ds(row0, RPT)])


_TC_R = 1000


def _tc_layer_body(relu, h_ref, p_ref, dp_ref, wo_ref, b_ref, wr_ref, o_ref):
    hb = h_ref[...]
    agg = hb + p_ref[0] + p_ref[1]
    deg = 1.0 + dp_ref[0] + dp_ref[1]
    agg = agg / jnp.maximum(deg, 1.0)
    out = (
        jnp.dot(agg, wo_ref[...], preferred_element_type=jnp.float32)
        + jnp.dot(hb, wr_ref[...], preferred_element_type=jnp.float32)
        + b_ref[...]
    )
    o_ref[...] = jnp.maximum(out, 0.0) if relu else out


def _tc_layer(h, p, dp, w_out, b_out, w_root, relu):
    grid = (N // _TC_R,)
    return pl.pallas_call(
        functools.partial(_tc_layer_body, relu),
        grid=grid,
        in_specs=[
            pl.BlockSpec((_TC_R, D), lambda i: (i, 0)),
            pl.BlockSpec((NC, _TC_R, D), lambda i: (0, i, 0)),
            pl.BlockSpec((NC, _TC_R, D), lambda i: (0, i, 0)),
            pl.BlockSpec((D, D), lambda i: (0, 0)),
            pl.BlockSpec((1, D), lambda i: (0, 0)),
            pl.BlockSpec((D, D), lambda i: (0, 0)),
        ],
        out_specs=pl.BlockSpec((_TC_R, D), lambda i: (i, 0)),
        out_shape=jax.ShapeDtypeStruct((N, D), jnp.float32),
    )(h, p, dp, w_out, b_out, w_root)


def kernel(x, edge_index, training,
           W_out0, b_out0, W_root0,
           W_out1, b_out1, W_root1,
           W_out2, b_out2, W_root2):
    del training
    src = edge_index[0]
    dst = edge_index[1]
    pad_e = E_PAD - E
    pad_ar = jnp.arange(pad_e, dtype=jnp.int32)
    src_p = jnp.concatenate([src, pad_ar % N])
    dst_p = jnp.concatenate([dst, N + pad_ar % (N_PAD - N)])
    src3 = src_p.reshape(NW, NB, B)
    dst3 = dst_p.reshape(NW, NB, B)
    zeros_tbl = jnp.zeros((N_PAD, D), jnp.float32)
    ones_blk = jnp.ones((B, D), jnp.float32)

    dp = _sc_deg_pass(dst3, zeros_tbl, ones_blk)

    b0 = b_out0.reshape(1, D)
    b1 = b_out1.reshape(1, D)
    b2 = b_out2.reshape(1, D)

    h = x
    p = _sc_edge_pass(h, src3, dst3, zeros_tbl)
    h = _tc_layer(h, p, dp, W_out0, b0, W_root0, relu=True)
    p = _sc_edge_pass(h, src3, dst3, zeros_tbl)
    h = _tc_layer(h, p, dp, W_out1, b1, W_root1, relu=True)
    p = _sc_edge_pass(h, src3, dst3, zeros_tbl)
    return _tc_layer(h, p, dp, W_out2, b2, W_root2, relu=False)


# speedup vs baseline: 21.8286x; 1.0359x over previous
import functools

import jax
import jax.numpy as jnp
from jax import lax
from jax.experimental import pallas as pl
from jax.experimental.pallas import tpu as pltpu
from jax.experimental.pallas import tpu_sc as plsc

N = 10000
E = 320000
D = 128

NC = 2
NS = 16
NW = NC * NS
B = 96
NB = 112
NBUF = 3
CH = 8
NCH = NB // CH

EPT = NB * B
E_PAD = NW * EPT
N_PAD = 10112
RPT = N_PAD // NS

_MESH = plsc.VectorSubcoreMesh(
    core_axis_name="c", subcore_axis_name="s", num_cores=NC, num_subcores=NS
)


@functools.partial(
    pl.kernel,
    out_type=jax.ShapeDtypeStruct((NC, N_PAD, D), jnp.float32),
    mesh=_MESH,
    scratch_types=[
        pltpu.VMEM((3, CH, B), jnp.int32),
        pltpu.VMEM((3, CH, B), jnp.int32),
        pltpu.VMEM((NBUF, B, D), jnp.float32),
        pltpu.VMEM_SHARED((N_PAD, D), jnp.float32),
        pltpu.SemaphoreType.DMA,
        pltpu.SemaphoreType.DMA,
        pltpu.SemaphoreType.DMA,
    ],
)
def _sc_edge_pass(h_hbm, src_hbm, dst_hbm, zeros_hbm, out_hbm,
                  src_ch, dst_ch, rows, acc_sh, isem, gsem, ssem):
    c = lax.axis_index("c")
    s = lax.axis_index("s")
    wid = s * NC + c
    row0 = pl.multiple_of(s * RPT, 8)
    pltpu.sync_copy(zeros_hbm.at[pl.ds(row0, RPT)], acc_sh.at[pl.ds(row0, RPT)])

    def i_start(k):
        slot = lax.rem(k, 3)
        pltpu.async_copy(src_hbm.at[wid, pl.ds(k * CH, CH)], src_ch.at[slot], isem)
        pltpu.async_copy(dst_hbm.at[wid, pl.ds(k * CH, CH)], dst_ch.at[slot], isem)

    def i_wait(k):
        slot = lax.rem(k, 3)
        pltpu.make_async_copy(
            src_hbm.at[wid, pl.ds(k * CH, CH)], src_ch.at[slot], isem).wait()
        pltpu.make_async_copy(
            dst_hbm.at[wid, pl.ds(k * CH, CH)], dst_ch.at[slot], isem).wait()

    def _slot_jj(i):
        k = lax.div(i, CH)
        return lax.rem(k, 3), lax.rem(i, CH)

    def g_start(i):
        slot, jj = _slot_jj(i)
        pltpu.async_copy(
            h_hbm.at[src_ch.at[slot, jj]], rows.at[lax.rem(i, NBUF)], gsem)

    def g_wait(i):
        slot, jj = _slot_jj(i)
        pltpu.make_async_copy(
            h_hbm.at[src_ch.at[slot, jj]], rows.at[lax.rem(i, NBUF)], gsem).wait()

    def s_start(i):
        slot, jj = _slot_jj(i)
        pltpu.async_copy(
            rows.at[lax.rem(i, NBUF)], acc_sh.at[dst_ch.at[slot, jj]], ssem,
            add=True)

    def s_wait(i):
        slot, jj = _slot_jj(i)
        pltpu.make_async_copy(
            rows.at[lax.rem(i, NBUF)], acc_sh.at[dst_ch.at[slot, jj]], ssem).wait()

    i_start(0)
    i_wait(0)
    i_start(1)
    for i in range(2):
        g_start(i)
    plsc.subcore_barrier()

    def body(i, carry):
        g_wait(i)
        s_start(i)
        pl.when(i >= 1)(lambda: s_wait(i - 1))

        def lookahead():
            nxt = i + 2

            def chunk_turn():
                k = lax.div(nxt, CH)
                i_wait(k)
                pl.when(k + 1 < NCH)(lambda: i_start(k + 1))

            pl.when(lax.rem(nxt, CH) == 0)(chunk_turn)
            g_start(nxt)

        pl.when(i + 2 < NB)(lookahead)
        return carry

    lax.fori_loop(0, NB, body, 0)
    s_wait(NB - 1)

    plsc.subcore_barrier()
    pltpu.sync_copy(acc_sh.at[pl.ds(row0, RPT)], out_hbm.at[c, pl.ds(row0, RPT)])


@functools.partial(
    pl.kernel,
    out_type=jax.ShapeDtypeStruct((NC, N_PAD, D), jnp.float32),
    mesh=_MESH,
    scratch_types=[
        pltpu.VMEM((NB, B), jnp.int32),
        pltpu.VMEM((B, D), jnp.float32),
        pltpu.VMEM_SHARED((N_PAD, D), jnp.float32),
        pltpu.SemaphoreType.DMA,
    ],
)
def _sc_deg_pass(dst_hbm, zeros_hbm, ones_hbm, out_hbm,
                 dst_all, ones_v, acc_sh, ssem):
    c = lax.axis_index("c")
    s = lax.axis_index("s")
    wid = s * NC + c
    row0 = pl.multiple_of(s * RPT, 8)
    pltpu.sync_copy(zeros_hbm.at[pl.ds(row0, RPT)], acc_sh.at[pl.ds(row0, RPT)])
    pltpu.sync_copy(dst_hbm.at[wid], dst_all)
    pltpu.sync_copy(ones_hbm, ones_v)
    plsc.subcore_barrier()

    def s_start(i):
        pltpu.async_copy(ones_v, acc_sh.at[dst_all.at[i]], ssem, add=True)

    def s_wait(i):
        pltpu.make_async_copy(ones_v, acc_sh.at[dst_all.at[i]], ssem).wait()

    def body(i, carry):
        s_start(i)
        pl.when(i >= 3)(lambda: s_wait(i - 3))
        return carry

    lax.fori_loop(0, NB, body, 0)
    for i in range(NB - 3, NB):
        s_wait(i)

    plsc.subcore_barrier()
    pltpu.sync_copy(acc_sh.at[pl.ds(row0, RPT)], out_hbm.at[c, pl.ds(row0, RPT)])


_TC_R = 1000


def _tc_layer_body(relu, h_ref, p_ref, dp_ref, wo_ref, b_ref, wr_ref, o_ref):
    hb = h_ref[...]
    agg = hb + p_ref[0] + p_ref[1]
    deg = 1.0 + dp_ref[0] + dp_ref[1]
    agg = agg / jnp.maximum(deg, 1.0)
    out = (
        jnp.dot(agg, wo_ref[...], preferred_element_type=jnp.float32)
        + jnp.dot(hb, wr_ref[...], preferred_element_type=jnp.float32)
        + b_ref[...]
    )
    o_ref[...] = jnp.maximum(out, 0.0) if relu else out


def _tc_layer(h, p, dp, w_out, b_out, w_root, relu):
    grid = (N // _TC_R,)
    return pl.pallas_call(
        functools.partial(_tc_layer_body, relu),
        grid=grid,
        in_specs=[
            pl.BlockSpec((_TC_R, D), lambda i: (i, 0)),
            pl.BlockSpec((NC, _TC_R, D), lambda i: (0, i, 0)),
            pl.BlockSpec((NC, _TC_R, D), lambda i: (0, i, 0)),
            pl.BlockSpec((D, D), lambda i: (0, 0)),
            pl.BlockSpec((1, D), lambda i: (0, 0)),
            pl.BlockSpec((D, D), lambda i: (0, 0)),
        ],
        out_specs=pl.BlockSpec((_TC_R, D), lambda i: (i, 0)),
        out_shape=jax.ShapeDtypeStruct((N, D), jnp.float32),
    )(h, p, dp, w_out, b_out, w_root)


def kernel(x, edge_index, training,
           W_out0, b_out0, W_root0,
           W_out1, b_out1, W_root1,
           W_out2, b_out2, W_root2):
    del training
    src = edge_index[0]
    dst = edge_index[1]
    pad_e = E_PAD - E
    pad_ar = jnp.arange(pad_e, dtype=jnp.int32)
    src_p = jnp.concatenate([src, pad_ar % N])
    dst_p = jnp.concatenate([dst, N + pad_ar % (N_PAD - N)])
    src3 = src_p.reshape(NW, NB, B)
    dst3 = dst_p.reshape(NW, NB, B)
    zeros_tbl = jnp.zeros((N_PAD, D), jnp.float32)
    ones_blk = jnp.ones((B, D), jnp.float32)

    dp = _sc_deg_pass(dst3, zeros_tbl, ones_blk)

    b0 = b_out0.reshape(1, D)
    b1 = b_out1.reshape(1, D)
    b2 = b_out2.reshape(1, D)

    h = x
    p = _sc_edge_pass(h, src3, dst3, zeros_tbl)
    h = _tc_layer(h, p, dp, W_out0, b0, W_root0, relu=True)
    p = _sc_edge_pass(h, src3, dst3, zeros_tbl)
    h = _tc_layer(h, p, dp, W_out1, b1, W_root1, relu=True)
    p = _sc_edge_pass(h, src3, dst3, zeros_tbl)
    return _tc_layer(h, p, dp, W_out2, b2, W_root2, relu=False)


# speedup vs baseline: 25.3881x; 1.1631x over previous
import functools

import jax
import jax.numpy as jnp
from jax import lax
from jax.experimental import pallas as pl
from jax.experimental.pallas import tpu as pltpu
from jax.experimental.pallas import tpu_sc as plsc

N = 10000
E = 320000
D = 128

NC = 2
NS = 16
NW = NC * NS
B = 96
NB = 112
NBUF = 3
CH = 8
NCH = NB // CH

EPT = NB * B
E_PAD = NW * EPT
N_PAD = 10240
RPT = N_PAD // NS
PK = N_PAD // 128
PKT = PK // NS

_MESH = plsc.VectorSubcoreMesh(
    core_axis_name="c", subcore_axis_name="s", num_cores=NC, num_subcores=NS
)


@functools.partial(
    pl.kernel,
    out_type=jax.ShapeDtypeStruct((NC, N_PAD, D), jnp.float32),
    mesh=_MESH,
    scratch_types=[
        pltpu.VMEM((3, CH, B), jnp.int32),
        pltpu.VMEM((3, CH, B), jnp.int32),
        pltpu.VMEM((NBUF, B, D), jnp.float32),
        pltpu.VMEM_SHARED((N_PAD, D), jnp.float32),
        pltpu.SemaphoreType.DMA,
        pltpu.SemaphoreType.DMA,
        pltpu.SemaphoreType.DMA,
    ],
)
def _sc_edge_pass(h_hbm, src_hbm, dst_hbm, zeros_hbm, out_hbm,
                  src_ch, dst_ch, rows, acc_sh, isem, gsem, ssem):
    c = lax.axis_index("c")
    s = lax.axis_index("s")
    wid = s * NC + c
    row0 = pl.multiple_of(s * RPT, 8)
    pltpu.sync_copy(zeros_hbm.at[pl.ds(row0, RPT)], acc_sh.at[pl.ds(row0, RPT)])

    def i_start(k):
        slot = lax.rem(k, 3)
        pltpu.async_copy(src_hbm.at[wid, pl.ds(k * CH, CH)], src_ch.at[slot], isem)
        pltpu.async_copy(dst_hbm.at[wid, pl.ds(k * CH, CH)], dst_ch.at[slot], isem)

    def i_wait(k):
        slot = lax.rem(k, 3)
        pltpu.make_async_copy(
            src_hbm.at[wid, pl.ds(k * CH, CH)], src_ch.at[slot], isem).wait()
        pltpu.make_async_copy(
            dst_hbm.at[wid, pl.ds(k * CH, CH)], dst_ch.at[slot], isem).wait()

    def _slot_jj(i):
        k = lax.div(i, CH)
        return lax.rem(k, 3), lax.rem(i, CH)

    def g_start(i):
        slot, jj = _slot_jj(i)
        pltpu.async_copy(
            h_hbm.at[src_ch.at[slot, jj]], rows.at[lax.rem(i, NBUF)], gsem)

    def g_wait(i):
        slot, jj = _slot_jj(i)
        pltpu.make_async_copy(
            h_hbm.at[src_ch.at[slot, jj]], rows.at[lax.rem(i, NBUF)], gsem).wait()

    def s_start(i):
        slot, jj = _slot_jj(i)
        pltpu.async_copy(
            rows.at[lax.rem(i, NBUF)], acc_sh.at[dst_ch.at[slot, jj]], ssem,
            add=True)

    def s_wait(i):
        slot, jj = _slot_jj(i)
        pltpu.make_async_copy(
            rows.at[lax.rem(i, NBUF)], acc_sh.at[dst_ch.at[slot, jj]], ssem).wait()

    i_start(0)
    i_wait(0)
    i_start(1)
    for i in range(2):
        g_start(i)
    plsc.subcore_barrier()

    def body(i, carry):
        g_wait(i)
        s_start(i)
        pl.when(i >= 1)(lambda: s_wait(i - 1))

        def lookahead():
            nxt = i + 2

            def chunk_turn():
                k = lax.div(nxt, CH)
                i_wait(k)
                pl.when(k + 1 < NCH)(lambda: i_start(k + 1))

            pl.when(lax.rem(nxt, CH) == 0)(chunk_turn)
            g_start(nxt)

        pl.when(i + 2 < NB)(lookahead)
        return carry

    lax.fori_loop(0, NB, body, 0)
    s_wait(NB - 1)

    plsc.subcore_barrier()
    pltpu.sync_copy(acc_sh.at[pl.ds(row0, RPT)], out_hbm.at[c, pl.ds(row0, RPT)])


@functools.partial(
    pl.kernel,
    out_type=jax.ShapeDtypeStruct((NC, PK, 128), jnp.float32),
    mesh=_MESH,
    scratch_types=[
        pltpu.VMEM((NB, B), jnp.int32),
        pltpu.VMEM((B,), jnp.float32),
        pltpu.VMEM_SHARED((N_PAD,), jnp.float32),
        pltpu.SemaphoreType.DMA,
    ],
)
def _sc_deg_pass(dst_hbm, zeros_hbm, out_hbm, dst_all, ones_v, acc_sh, ssem):
    c = lax.axis_index("c")
    s = lax.axis_index("s")
    wid = s * NC + c
    el0 = pl.multiple_of(s * RPT, 128)
    pltpu.sync_copy(zeros_hbm.at[pl.ds(el0, RPT)], acc_sh.at[pl.ds(el0, RPT)])
    pltpu.sync_copy(dst_hbm.at[wid], dst_all)
    for j in range(B // 16):
        ones_v[pl.ds(16 * j, 16)] = jnp.full((16,), 1.0, jnp.float32)
    plsc.subcore_barrier()

    def s_start(i):
        pltpu.async_copy(ones_v, acc_sh.at[dst_all.at[i]], ssem, add=True)

    def s_wait(i):
        pltpu.make_async_copy(ones_v, acc_sh.at[dst_all.at[i]], ssem).wait()

    def body(i, carry):
        s_start(i)
        pl.when(i >= 3)(lambda: s_wait(i - 3))
        return carry

    lax.fori_loop(0, NB, body, 0)
    for i in range(NB - 3, NB):
        s_wait(i)

    plsc.subcore_barrier()
    for jj in range(PKT):
        j = s * PKT + jj
        pltpu.sync_copy(acc_sh.at[pl.ds(pl.multiple_of(j * 128, 128), 128)],
                        out_hbm.at[c, j])


_TC_R = 1280
_TC_PK = _TC_R // 128


def _tc_layer_body(relu, h_ref, p_ref, dpq_ref, a_ref, m_ref,
                   wo_ref, b_ref, wr_ref, o_ref):
    hb = h_ref[...]
    agg = hb + p_ref[0] + p_ref[1]
    pk = dpq_ref[0] + dpq_ref[1]
    deg1 = jnp.dot(a_ref[...], pk, preferred_element_type=jnp.float32)
    dcol = jnp.sum(deg1 * m_ref[...], axis=1, keepdims=True)
    agg = agg / jnp.maximum(1.0 + dcol, 1.0)
    out = (
        jnp.dot(agg, wo_ref[...], preferred_element_type=jnp.float32)
        + jnp.dot(hb, wr_ref[...], preferred_element_type=jnp.float32)
        + b_ref[...]
    )
    o_ref[...] = jnp.maximum(out, 0.0) if relu else out


def _tc_layer(h, p, dpq, a_sel, m_sel, w_out, b_out, w_root, relu):
    grid = (N_PAD // _TC_R,)
    return pl.pallas_call(
        functools.partial(_tc_layer_body, relu),
        grid=grid,
        in_specs=[
            pl.BlockSpec((_TC_R, D), lambda i: (i, 0)),
            pl.BlockSpec((NC, _TC_R, D), lambda i: (0, i, 0)),
            pl.BlockSpec((NC, PK, 128), lambda i: (0, 0, 0)),
            pl.BlockSpec((_TC_R, PK), lambda i: (i, 0)),
            pl.BlockSpec((_TC_R, 128), lambda i: (0, 0)),
            pl.BlockSpec((D, D), lambda i: (0, 0)),
            pl.BlockSpec((1, D), lambda i: (0, 0)),
            pl.BlockSpec((D, D), lambda i: (0, 0)),
        ],
        out_specs=pl.BlockSpec((_TC_R, D), lambda i: (i, 0)),
        out_shape=jax.ShapeDtypeStruct((N_PAD, D), jnp.float32),
    )(h, p, dpq, a_sel, m_sel, w_out, b_out, w_root)


def kernel(x, edge_index, training,
           W_out0, b_out0, W_root0,
           W_out1, b_out1, W_root1,
           W_out2, b_out2, W_root2):
    del training
    src = edge_index[0]
    dst = edge_index[1]
    pad_e = E_PAD - E
    pad_ar = jnp.arange(pad_e, dtype=jnp.int32)
    src_p = jnp.concatenate([src, pad_ar % N])
    dst_p = jnp.concatenate([dst, N + pad_ar % (N_PAD - N)])
    src3 = src_p.reshape(NW, NB, B)
    dst3 = dst_p.reshape(NW, NB, B)
    zeros_tbl = jnp.zeros((N_PAD, D), jnp.float32)
    zeros_1d = jnp.zeros((N_PAD,), jnp.float32)
    a_sel = (jnp.arange(N_PAD)[:, None] // 128
             == jnp.arange(PK)[None, :]).astype(jnp.float32)
    m_sel = (jnp.arange(_TC_R)[:, None] % 128
             == jnp.arange(128)[None, :]).astype(jnp.float32)

    dpq = _sc_deg_pass(dst3, zeros_1d)

    b0 = b_out0.reshape(1, D)
    b1 = b_out1.reshape(1, D)
    b2 = b_out2.reshape(1, D)

    h = jnp.concatenate([x, jnp.zeros((N_PAD - N, D), jnp.float32)])
    p = _sc_edge_pass(h, src3, dst3, zeros_tbl)
    h = _tc_layer(h, p, dpq, a_sel, m_sel, W_out0, b0, W_root0, relu=True)
    p = _sc_edge_pass(h, src3, dst3, zeros_tbl)
    h = _tc_layer(h, p, dpq, a_sel, m_sel, W_out1, b1, W_root1, relu=True)
    p = _sc_edge_pass(h, src3, dst3, zeros_tbl)
    out = _tc_layer(h, p, dpq, a_sel, m_sel, W_out2, b2, W_root2, relu=False)
    return out[:N]
